# Initial kernel scaffold; baseline (speedup 1.0000x reference)
import jax
import jax.numpy as jnp
from jax.experimental import pallas as pl


def kernel(x_chemical, x_gene, edge_index_cg, edge_index_gc, edge_attr_cg, edge_attr_gc, params):
    raise NotImplementedError("write your pallas kernel here")


# speedup vs baseline: 4.0248x; 4.0248x over previous
import functools
import math

import jax
import jax.numpy as jnp
from jax import lax
from jax.experimental import pallas as pl
from jax.experimental.pallas import tpu as pltpu
from jax.experimental.pallas import tpu_sc as plsc

N = 10000
D = 128
E = 160000
HEADS = 4
HD = D // HEADS
EAD = 32
NAT = 8
NAS = 4

NC = 2
NS = 16
NW = NC * NS
C = 64
EP = 5120
E_PAD = NW * EP
NCHUNK = EP // C
NUM_ROWS = 10240
DEN_ROW0 = 10240
ACC_ROWS = 10752
RPT = ACC_ROWS // NS
ZR = 16
LANES = 16


def _tc_prep(x_c, x_g, wsc, bsc, wdc, bdc, wsg, bsg, wdg, bdg):
    blk = 1000
    grid = (N // blk,)

    def body(xc, xg, wsc_r, bsc_r, wdc_r, bdc_r, wsg_r, bsg_r, wdg_r, bdg_r,
             o1, o2, o3, o4):
        xc_v = xc[...]
        xg_v = xg[...]
        o1[...] = jnp.dot(xc_v, wsc_r[...], preferred_element_type=jnp.float32) + bsc_r[...]
        o2[...] = jnp.dot(xg_v, wdc_r[...], preferred_element_type=jnp.float32) + bdc_r[...]
        o3[...] = jnp.dot(xg_v, wsg_r[...], preferred_element_type=jnp.float32) + bsg_r[...]
        o4[...] = jnp.dot(xc_v, wdg_r[...], preferred_element_type=jnp.float32) + bdg_r[...]

    row_spec = pl.BlockSpec((blk, D), lambda i: (i, 0))
    w_spec = pl.BlockSpec((D, D), lambda i: (0, 0))
    b_spec = pl.BlockSpec((1, D), lambda i: (0, 0))
    out_sd = jax.ShapeDtypeStruct((N, D), jnp.float32)
    return pl.pallas_call(
        body,
        grid=grid,
        in_specs=[row_spec, row_spec, w_spec, b_spec, w_spec, b_spec,
                  w_spec, b_spec, w_spec, b_spec],
        out_specs=[row_spec, row_spec, row_spec, row_spec],
        out_shape=[out_sd, out_sd, out_sd, out_sd],
    )(x_c, x_g, wsc, bsc.reshape(1, D), wdc, bdc.reshape(1, D),
      wsg, bsg.reshape(1, D), wdg, bdg.reshape(1, D))


def _tc_gate(cat_tab, wg_cg, bg_cg, wg_gc, bg_gc):
    def body(ct, w1, b1, w2, b2, o1, o2):
        ct_v = ct[...]
        o1[...] = jax.nn.sigmoid(
            jnp.dot(ct_v, w1[...], preferred_element_type=jnp.float32) + b1[...])
        o2[...] = jax.nn.sigmoid(
            jnp.dot(ct_v, w2[...], preferred_element_type=jnp.float32) + b2[...])

    out_sd = jax.ShapeDtypeStruct((NAT * NAS, D), jnp.float32)
    return pl.pallas_call(body, out_shape=[out_sd, out_sd])(
        cat_tab, wg_cg, bg_cg.reshape(1, D), wg_gc, bg_gc.reshape(1, D))


def _tc_final(num_chem, den_chem, num_gene, den_gene, w_oc, b_oc, w_og, b_og):
    blk = 1000
    grid = (N // blk,)

    def body(nc_r, dc_r, ng_r, dg_r, woc, boc, wog, bog, oc, og):
        def merge(n, dp):
            num = n[0] + n[1]
            den = dp[0] + dp[1] + 1e-16
            denb = jnp.concatenate(
                [jnp.broadcast_to(den[:, h:h + 1], (blk, HD)) for h in range(HEADS)],
                axis=1)
            return num / denb

        agg_c = merge(nc_r[...], dc_r[...])
        agg_g = merge(ng_r[...], dg_r[...])
        oc[...] = jnp.dot(agg_c, woc[...], preferred_element_type=jnp.float32) + boc[...]
        og[...] = jnp.dot(agg_g, wog[...], preferred_element_type=jnp.float32) + bog[...]

    n_spec = pl.BlockSpec((NC, blk, D), lambda i: (0, i, 0))
    d_spec = pl.BlockSpec((NC, blk, HEADS), lambda i: (0, i, 0))
    w_spec = pl.BlockSpec((D, D), lambda i: (0, 0))
    b_spec = pl.BlockSpec((1, D), lambda i: (0, 0))
    o_spec = pl.BlockSpec((blk, D), lambda i: (i, 0))
    out_sd = jax.ShapeDtypeStruct((N, D), jnp.float32)
    return pl.pallas_call(
        body,
        grid=grid,
        in_specs=[n_spec, d_spec, n_spec, d_spec, w_spec, b_spec, w_spec, b_spec],
        out_specs=[o_spec, o_spec],
        out_shape=[out_sd, out_sd],
    )(num_chem, den_chem, num_gene, den_gene,
      w_oc, b_oc.reshape(1, D), w_og, b_og.reshape(1, D))


_INV_SQRT_HD = 1.0 / math.sqrt(HD)


def _splat_sum(v):
    lane = lax.iota(jnp.int32, LANES)
    for sh in (8, 4, 2, 1):
        v = v + jnp.take_along_axis(v, lane ^ sh, axis=0)
    return v


def _sc_conv(h_src, h_dst, gate_tab, attn_flat, src_idx, dst_idx, gcode):
    mesh = plsc.VectorSubcoreMesh(core_axis_name="c", subcore_axis_name="s")

    @functools.partial(
        pl.kernel,
        out_type=(jax.ShapeDtypeStruct((NC, NUM_ROWS, D), jnp.float32),
                  jax.ShapeDtypeStruct((NC, (ACC_ROWS - DEN_ROW0), D), jnp.float32)),
        mesh=mesh,
        compiler_params=pltpu.CompilerParams(needs_layout_passes=False),
        scratch_types=[
            pltpu.VMEM((C, D), jnp.float32),
            pltpu.VMEM((C, D), jnp.float32),
            pltpu.VMEM((C, D), jnp.float32),
            pltpu.VMEM((C, D), jnp.float32),
            pltpu.VMEM((C, D), jnp.float32),
            pltpu.VMEM((C,), jnp.int32),
            pltpu.VMEM((C,), jnp.int32),
            pltpu.VMEM((C,), jnp.int32),
            pltpu.VMEM((C,), jnp.int32),
            pltpu.VMEM((D,), jnp.float32),
            pltpu.VMEM((ZR, D), jnp.float32),
            pltpu.VMEM_SHARED((ACC_ROWS, D), jnp.float32),
            pltpu.SemaphoreType.DMA,
            pltpu.SemaphoreType.DMA,
            pltpu.SemaphoreType.DMA,
        ],
    )
    def k(hs_hbm, hd_hbm, gt_hbm, at_hbm, si_hbm, di_hbm, gc_hbm,
          onum_hbm, oden_hbm,
          srows, drows, grows, orows, dorows, sidx, didx, gidx, d2idx,
          attnv, zbuf, acc, sem_s, sem_d, sem_g):
        cid = lax.axis_index("c")
        sid = lax.axis_index("s")
        wid = sid * NC + cid
        ebase = wid * EP
        rbase = sid * RPT
        lane = lax.iota(jnp.int32, LANES)
        zeros = jnp.zeros((LANES,), jnp.float32)

        pltpu.sync_copy(at_hbm, attnv)

        def zrow(r, carry):
            for k9 in range(D // LANES):
                zbuf[r, pl.ds(LANES * k9, LANES)] = zeros
            return carry

        lax.fori_loop(0, ZR, zrow, 0)

        def zcopy(j, carry):
            pltpu.sync_copy(zbuf, acc.at[pl.ds(rbase + j * ZR, ZR)])
            return carry

        lax.fori_loop(0, RPT // ZR, zcopy, 0)
        plsc.subcore_barrier()

        def chunk(ci, carry):
            base = ebase + ci * C
            pltpu.sync_copy(si_hbm.at[pl.ds(base, C)], sidx)
            pltpu.sync_copy(di_hbm.at[pl.ds(base, C)], didx)
            pltpu.sync_copy(gc_hbm.at[pl.ds(base, C)], gidx)
            cp_s = pltpu.async_copy(hs_hbm.at[sidx], srows, sem_s)
            cp_d = pltpu.async_copy(hd_hbm.at[didx], drows, sem_d)
            cp_g = pltpu.async_copy(gt_hbm.at[gidx], grows, sem_g)

            def dfill(j, carry2):
                dv = didx[pl.ds(LANES * j, LANES)]
                d2idx[pl.ds(LANES * j, LANES)] = (
                    DEN_ROW0 + lax.shift_right_logical(dv, 5))
                return carry2

            lax.fori_loop(0, C // LANES, dfill, 0)
            cp_s.wait()
            cp_d.wait()
            cp_g.wait()

            def edge(e, ecarry):
                s = [srows[e, pl.ds(LANES * k9, LANES)] for k9 in range(8)]
                d = [drows[e, pl.ds(LANES * k9, LANES)] for k9 in range(8)]
                g = [grows[e, pl.ds(LANES * k9, LANES)] for k9 in range(8)]
                m = [(s[k9] + d[k9]) * g[k9] for k9 in range(8)]
                u = [s[k9] * d[k9] * _INV_SQRT_HD
                     + m[k9] * attnv[pl.ds(LANES * k9, LANES)]
                     for k9 in range(8)]
                exb = [jnp.exp(_splat_sum(u[2 * h] + u[2 * h + 1]))
                       for h in range(HEADS)]
                for k9 in range(8):
                    orows[e, pl.ds(LANES * k9, LANES)] = m[k9] * exb[k9 // 2]
                    dorows[e, pl.ds(LANES * k9, LANES)] = zeros
                exvec = jnp.where(lane == 0, exb[0], jnp.float32(0.0))
                exvec = jnp.where(lane == 1, exb[1], exvec)
                exvec = jnp.where(lane == 2, exb[2], exvec)
                exvec = jnp.where(lane == 3, exb[3], exvec)
                esplat = jnp.full((LANES,), e, jnp.int32)
                dsplat = plsc.load_gather(didx, [esplat])
                loff = lax.shift_left(lax.bitwise_and(dsplat, 31), 2)
                plsc.store_scatter(dorows, [esplat, loff + lane], exvec,
                                   mask=lane < HEADS)
                return ecarry

            lax.fori_loop(0, C, edge, 0)
            pltpu.sync_copy(orows, acc.at[didx], add=True)
            pltpu.sync_copy(dorows, acc.at[d2idx], add=True)
            return carry

        lax.fori_loop(0, NCHUNK, chunk, 0)
        plsc.subcore_barrier()

        nrpt = NUM_ROWS // NS
        pltpu.sync_copy(acc.at[pl.ds(sid * nrpt, nrpt)],
                        onum_hbm.at[cid, pl.ds(sid * nrpt, nrpt)])
        drpt = (ACC_ROWS - DEN_ROW0) // NS
        pltpu.sync_copy(acc.at[pl.ds(DEN_ROW0 + sid * drpt, drpt)],
                        oden_hbm.at[cid, pl.ds(sid * drpt, drpt)])

    return k(h_src, h_dst, gate_tab, attn_flat, src_idx, dst_idx, gcode)


def _pad_idx(src, dst, gc):
    pad = E_PAD - E
    src = jnp.concatenate([src, jnp.zeros((pad,), jnp.int32)])
    dst = jnp.concatenate([dst, jnp.full((pad,), N, jnp.int32)])
    gc = jnp.concatenate([gc, jnp.zeros((pad,), jnp.int32)])
    return src, dst, gc


def kernel(x_chemical, x_gene, edge_index_cg, edge_index_gc, edge_attr_cg,
           edge_attr_gc, params):
    p = params
    h_src_cg, h_dst_cg, h_src_gc, h_dst_gc = _tc_prep(
        x_chemical, x_gene,
        p["W_src_cg"], p["b_src_cg"], p["W_dst_cg"], p["b_dst_cg"],
        p["W_src_gc"], p["b_src_gc"], p["W_dst_gc"], p["b_dst_gc"])

    ate = p["action_type_emb"]
    ase = p["action_subject_emb"]
    te = jnp.repeat(ate, NAS, axis=0)
    se = jnp.tile(ase, (NAT, 1))
    cat_tab = jnp.concatenate([te, se], axis=1)
    gate_cg, gate_gc = _tc_gate(cat_tab, p["W_gate_cg"], p["b_gate_cg"],
                                p["W_gate_gc"], p["b_gate_gc"])

    attn_cg = p["attn_cg"].reshape(D)
    attn_gc = p["attn_gc"].reshape(D)

    def prep_idx(ei, ea):
        src = ei[0].astype(jnp.int32)
        dst = ei[1].astype(jnp.int32)
        gc = (ea[:, 0] * NAS + ea[:, 1]).astype(jnp.int32)
        return _pad_idx(src, dst, gc)

    s_cg, d_cg, g_cg = prep_idx(edge_index_cg, edge_attr_cg)
    s_gc, d_gc, g_gc = prep_idx(edge_index_gc, edge_attr_gc)

    num_gene, den_gene = _sc_conv(h_src_cg, h_dst_cg, gate_cg, attn_cg,
                                  s_cg, d_cg, g_cg)
    num_chem, den_chem = _sc_conv(h_src_gc, h_dst_gc, gate_gc, attn_gc,
                                  s_gc, d_gc, g_gc)
    dn = (ACC_ROWS - DEN_ROW0) * D // HEADS
    den_gene = den_gene.reshape(NC, dn, HEADS)
    den_chem = den_chem.reshape(NC, dn, HEADS)

    out_chemical, out_gene = _tc_final(
        num_chem, den_chem, num_gene, den_gene,
        p["W_out_chemical"], p["b_out_chemical"],
        p["W_out_gene"], p["b_out_gene"])
    return (out_chemical, out_gene)
